# baseline (speedup 1.0000x reference)
import functools

import jax
import jax.numpy as jnp
from jax import lax
from jax.experimental import pallas as pl
from jax.experimental.pallas import tpu as pltpu
from jax.experimental.pallas import tpu_sc as plsc

L = 16
CPAD = 1008
WCOLS = 64
CHUNK = 128
NBUF = 4
N_ROWS = 16384
NW = 32


def _sc_body(
    f_hbm,
    t_hbm,
    w_hbm,
    out_hbm,
    f_buf0,
    f_buf1,
    f_buf2,
    f_buf3,
    t_buf0,
    t_buf1,
    t_buf2,
    t_buf3,
    w_buf,
    s_flat,
    cnt,
    out_buf,
    f_sem0,
    f_sem1,
    f_sem2,
    f_sem3,
    t_sem0,
    t_sem1,
    t_sem2,
    t_sem3,
):
    c = lax.axis_index("c")
    s = lax.axis_index("s")
    wid = s * 2 + c

    f_bufs = (f_buf0, f_buf1, f_buf2, f_buf3)
    t_bufs = (t_buf0, t_buf1, t_buf2, t_buf3)
    f_sems = (f_sem0, f_sem1, f_sem2, f_sem3)
    t_sems = (t_sem0, t_sem1, t_sem2, t_sem3)

    zero = jnp.zeros((L,), jnp.float32)
    one = jnp.ones((L,), jnp.float32)
    col_iota = lax.iota(jnp.int32, L)
    lane0 = col_iota == 0
    nch = N_ROWS // CHUNK

    def zc(k, x):
        cnt[pl.ds(k * L, L)] = zero
        return x

    lax.fori_loop(0, CPAD // L, zc, 0)

    f2acc = zero
    cross = zero
    nw2 = zero

    for p in range(2):
        col0 = p * (NW * WCOLS) + wid * WCOLS

        def f_slice(g):
            return f_hbm.at[pl.ds(g * CHUNK, CHUNK), pl.ds(col0, WCOLS)]

        def t_slice(g):
            return t_hbm.at[pl.ds(g * CHUNK, CHUNK)]

        def zs(k, x):
            for u in range(4):
                s_flat[pl.ds((k * 4 + u) * L, L)] = zero
            return x

        lax.fori_loop(0, CPAD * WCOLS // (4 * L), zs, 0)

        for b in range(NBUF):
            pltpu.async_copy(f_slice(b), f_bufs[b], f_sems[b])
            pltpu.async_copy(t_slice(b), t_bufs[b], t_sems[b])

        nj = WCOLS // L

        def compute_chunk(fb, tb, a):
            def m_body(m, a):
                tv = tb[pl.ds(m * L, L)]
                r0 = m * L
                vs = [fb[r0, pl.ds(j * L, L)] for j in range(nj)]
                for l in range(L):
                    cur = vs
                    if l < L - 1:
                        vs = [
                            fb[r0 + l + 1, pl.ds(j * L, L)]
                            for j in range(nj)
                        ]
                    tsp = tv.at[jnp.full((L,), l, jnp.int32)].get(
                        mode="promise_in_bounds"
                    )
                    tbase = tsp * WCOLS
                    if p == 0:
                        plsc.addupdate_scatter(
                            cnt, [tsp], one, mask=lane0
                        )
                    for j in range(nj):
                        a = a + cur[j] * cur[j]
                        plsc.addupdate_scatter(
                            s_flat, [tbase + (col_iota + j * L)], cur[j]
                        )
                return a

            return lax.fori_loop(0, CHUNK // L, m_body, a)

        def pair_body(gp, acc):
            for b in range(NBUF):
                g = gp * NBUF + b
                pltpu.make_async_copy(
                    f_slice(g), f_bufs[b], f_sems[b]
                ).wait()
                pltpu.make_async_copy(
                    t_slice(g), t_bufs[b], t_sems[b]
                ).wait()
                acc = compute_chunk(f_bufs[b], t_bufs[b], acc)
                g2 = jnp.minimum(g + NBUF, nch - 1)
                pltpu.async_copy(f_slice(g2), f_bufs[b], f_sems[b])
                pltpu.async_copy(t_slice(g2), t_bufs[b], t_sems[b])
            return acc

        f2acc = lax.fori_loop(0, nch // NBUF, pair_body, f2acc)
        for b in range(NBUF):
            pltpu.make_async_copy(f_slice(0), f_bufs[b], f_sems[b]).wait()
            pltpu.make_async_copy(t_slice(0), t_bufs[b], t_sems[b]).wait()

        def ep_body(g, carry):
            cr, nw = carry
            pltpu.sync_copy(
                w_hbm.at[pl.ds(g * L, L), pl.ds(col0, WCOLS)], w_buf
            )
            for l in range(L):
                cbase = (g * L + l) * WCOLS
                nsp = plsc.load_gather(
                    cnt, [jnp.full((L,), g * L + l, jnp.int32)]
                )
                wsq = zero
                for j in range(WCOLS // L):
                    sv = s_flat[pl.ds(cbase + j * L, L)]
                    wv = w_buf[l, pl.ds(j * L, L)]
                    cr = cr + sv * wv
                    wsq = wsq + wv * wv
                nw = nw + nsp * wsq
            return (cr, nw)

        cross, nw2 = lax.fori_loop(0, CPAD // L, ep_body, (cross, nw2))

    out_buf[0, :] = f2acc
    out_buf[1, :] = cross
    out_buf[2, :] = nw2
    pltpu.sync_copy(out_buf, out_hbm.at[c, s])


def kernel(feature, target, weight):
    n, d = feature.shape
    nc = weight.shape[0]

    w_pad = jnp.zeros((CPAD, d), jnp.float32).at[:nc].set(weight)

    mesh = plsc.VectorSubcoreMesh(core_axis_name="c", subcore_axis_name="s")
    out = pl.kernel(
        _sc_body,
        mesh=mesh,
        out_type=jax.ShapeDtypeStruct((2, 16, 3, L), jnp.float32),
        compiler_params=pltpu.CompilerParams(
            use_tc_tiling_on_sc=False, needs_layout_passes=False
        ),
        scratch_types=[
            pltpu.VMEM((CHUNK, WCOLS), jnp.float32),
            pltpu.VMEM((CHUNK, WCOLS), jnp.float32),
            pltpu.VMEM((CHUNK, WCOLS), jnp.float32),
            pltpu.VMEM((CHUNK, WCOLS), jnp.float32),
            pltpu.VMEM((CHUNK,), jnp.int32),
            pltpu.VMEM((CHUNK,), jnp.int32),
            pltpu.VMEM((CHUNK,), jnp.int32),
            pltpu.VMEM((CHUNK,), jnp.int32),
            pltpu.VMEM((L, WCOLS), jnp.float32),
            pltpu.VMEM((CPAD * WCOLS,), jnp.float32),
            pltpu.VMEM((CPAD,), jnp.float32),
            pltpu.VMEM((3, L), jnp.float32),
            pltpu.SemaphoreType.DMA,
            pltpu.SemaphoreType.DMA,
            pltpu.SemaphoreType.DMA,
            pltpu.SemaphoreType.DMA,
            pltpu.SemaphoreType.DMA,
            pltpu.SemaphoreType.DMA,
            pltpu.SemaphoreType.DMA,
            pltpu.SemaphoreType.DMA,
        ],
    )(feature, target, w_pad)

    sf2 = jnp.sum(out[:, :, 0, :])
    cross = jnp.sum(out[:, :, 1, :])
    nw2 = jnp.sum(out[:, :, 2, :])
    return (sf2 - 2.0 * cross + nw2) / (n * d)


# speedup vs baseline: 1.1926x; 1.1926x over previous
import functools

import jax
import jax.numpy as jnp
from jax import lax
from jax.experimental import pallas as pl
from jax.experimental.pallas import tpu as pltpu
from jax.experimental.pallas import tpu_sc as plsc

L = 16
CPAD = 1008
WCOLS = 64
CHUNK = 128
NBUF = 4
N_ROWS = 16384
ROW0 = 11264
SC_ROWS = N_ROWS - ROW0
NW = 32


def _sc_body(
    f_hbm,
    t_hbm,
    w_hbm,
    out_hbm,
    f_buf0,
    f_buf1,
    f_buf2,
    f_buf3,
    t_buf0,
    t_buf1,
    t_buf2,
    t_buf3,
    w_buf,
    s_flat,
    cnt,
    out_buf,
    f_sem0,
    f_sem1,
    f_sem2,
    f_sem3,
    t_sem0,
    t_sem1,
    t_sem2,
    t_sem3,
):
    c = lax.axis_index("c")
    s = lax.axis_index("s")
    wid = s * 2 + c

    f_bufs = (f_buf0, f_buf1, f_buf2, f_buf3)
    t_bufs = (t_buf0, t_buf1, t_buf2, t_buf3)
    f_sems = (f_sem0, f_sem1, f_sem2, f_sem3)
    t_sems = (t_sem0, t_sem1, t_sem2, t_sem3)

    zero = jnp.zeros((L,), jnp.float32)
    one = jnp.ones((L,), jnp.float32)
    col_iota = lax.iota(jnp.int32, L)
    lane0 = col_iota == 0
    nch = SC_ROWS // CHUNK

    def zc(k, x):
        cnt[pl.ds(k * L, L)] = zero
        return x

    lax.fori_loop(0, CPAD // L, zc, 0)

    f2acc = zero
    cross = zero
    nw2 = zero

    for p in range(2):
        col0 = p * (NW * WCOLS) + wid * WCOLS

        def f_slice(g):
            return f_hbm.at[
                pl.ds(ROW0 + g * CHUNK, CHUNK), pl.ds(col0, WCOLS)
            ]

        def t_slice(g):
            return t_hbm.at[pl.ds(ROW0 + g * CHUNK, CHUNK)]

        def zs(k, x):
            for u in range(4):
                s_flat[pl.ds((k * 4 + u) * L, L)] = zero
            return x

        lax.fori_loop(0, CPAD * WCOLS // (4 * L), zs, 0)

        for b in range(NBUF):
            pltpu.async_copy(f_slice(b), f_bufs[b], f_sems[b])
            pltpu.async_copy(t_slice(b), t_bufs[b], t_sems[b])

        nj = WCOLS // L

        def compute_chunk(fb, tb, a):
            def m_body(m, a):
                tv = tb[pl.ds(m * L, L)]
                r0 = m * L
                vs = [fb[r0, pl.ds(j * L, L)] for j in range(nj)]
                for l in range(L):
                    cur = vs
                    if l < L - 1:
                        vs = [
                            fb[r0 + l + 1, pl.ds(j * L, L)]
                            for j in range(nj)
                        ]
                    tsp = tv.at[jnp.full((L,), l, jnp.int32)].get(
                        mode="promise_in_bounds"
                    )
                    tbase = tsp * WCOLS
                    if p == 0:
                        plsc.addupdate_scatter(
                            cnt, [tsp], one, mask=lane0
                        )
                    for j in range(nj):
                        a = a + cur[j] * cur[j]
                        plsc.addupdate_scatter(
                            s_flat, [tbase + (col_iota + j * L)], cur[j]
                        )
                return a

            return lax.fori_loop(0, CHUNK // L, m_body, a)

        def pair_body(gp, acc):
            for b in range(NBUF):
                g = gp * NBUF + b
                pltpu.make_async_copy(
                    f_slice(g), f_bufs[b], f_sems[b]
                ).wait()
                pltpu.make_async_copy(
                    t_slice(g), t_bufs[b], t_sems[b]
                ).wait()
                acc = compute_chunk(f_bufs[b], t_bufs[b], acc)
                g2 = jnp.minimum(g + NBUF, nch - 1)
                pltpu.async_copy(f_slice(g2), f_bufs[b], f_sems[b])
                pltpu.async_copy(t_slice(g2), t_bufs[b], t_sems[b])
            return acc

        f2acc = lax.fori_loop(0, nch // NBUF, pair_body, f2acc)
        for b in range(NBUF):
            pltpu.make_async_copy(f_slice(0), f_bufs[b], f_sems[b]).wait()
            pltpu.make_async_copy(t_slice(0), t_bufs[b], t_sems[b]).wait()

        def ep_body(g, carry):
            cr, nw = carry
            pltpu.sync_copy(
                w_hbm.at[pl.ds(g * L, L), pl.ds(col0, WCOLS)], w_buf
            )
            for l in range(L):
                cbase = (g * L + l) * WCOLS
                nsp = plsc.load_gather(
                    cnt, [jnp.full((L,), g * L + l, jnp.int32)]
                )
                wsq = zero
                for j in range(WCOLS // L):
                    sv = s_flat[pl.ds(cbase + j * L, L)]
                    wv = w_buf[l, pl.ds(j * L, L)]
                    cr = cr + sv * wv
                    wsq = wsq + wv * wv
                nw = nw + nsp * wsq
            return (cr, nw)

        cross, nw2 = lax.fori_loop(0, CPAD // L, ep_body, (cross, nw2))

    out_buf[0, :] = f2acc
    out_buf[1, :] = cross
    out_buf[2, :] = nw2
    pltpu.sync_copy(out_buf, out_hbm.at[c, s])


def _sc_call(feature, target, weight):
    n, d = feature.shape
    nc = weight.shape[0]

    w_pad = jnp.zeros((CPAD, d), jnp.float32).at[:nc].set(weight)

    mesh = plsc.VectorSubcoreMesh(core_axis_name="c", subcore_axis_name="s")
    out = pl.kernel(
        _sc_body,
        mesh=mesh,
        out_type=jax.ShapeDtypeStruct((2, 16, 3, L), jnp.float32),
        compiler_params=pltpu.CompilerParams(
            use_tc_tiling_on_sc=False, needs_layout_passes=False
        ),
        scratch_types=[
            pltpu.VMEM((CHUNK, WCOLS), jnp.float32),
            pltpu.VMEM((CHUNK, WCOLS), jnp.float32),
            pltpu.VMEM((CHUNK, WCOLS), jnp.float32),
            pltpu.VMEM((CHUNK, WCOLS), jnp.float32),
            pltpu.VMEM((CHUNK,), jnp.int32),
            pltpu.VMEM((CHUNK,), jnp.int32),
            pltpu.VMEM((CHUNK,), jnp.int32),
            pltpu.VMEM((CHUNK,), jnp.int32),
            pltpu.VMEM((L, WCOLS), jnp.float32),
            pltpu.VMEM((CPAD * WCOLS,), jnp.float32),
            pltpu.VMEM((CPAD,), jnp.float32),
            pltpu.VMEM((3, L), jnp.float32),
            pltpu.SemaphoreType.DMA,
            pltpu.SemaphoreType.DMA,
            pltpu.SemaphoreType.DMA,
            pltpu.SemaphoreType.DMA,
            pltpu.SemaphoreType.DMA,
            pltpu.SemaphoreType.DMA,
            pltpu.SemaphoreType.DMA,
            pltpu.SemaphoreType.DMA,
        ],
    )(feature, target, w_pad)

    sf2 = jnp.sum(out[:, :, 0, :])
    cross = jnp.sum(out[:, :, 1, :])
    nw2 = jnp.sum(out[:, :, 2, :])
    return sf2 - 2.0 * cross + nw2


def _tc_body(t_ref, f_ref, w_ref, o_ref, *, rows_per_block, cpad):
    pid = pl.program_id(0)

    @pl.when(pid == 0)
    def _():
        o_ref[0, 0] = 0.0

    t_col = t_ref[0]
    iota = lax.broadcasted_iota(jnp.int32, (rows_per_block, cpad), 1)
    onehot = (t_col == iota).astype(jnp.bfloat16)
    g = jnp.dot(onehot, w_ref[...], preferred_element_type=jnp.float32)
    diff = f_ref[...] - g
    o_ref[0, 0] += jnp.sum(diff * diff)


def _tc_call(feature, target, weight):
    n, d = feature.shape
    nc = weight.shape[0]
    cpad = 1024
    rows_per_block = 512
    nb = ROW0 // rows_per_block

    w_bf = (
        jnp.zeros((cpad, d), jnp.bfloat16)
        .at[:nc]
        .set(weight.astype(jnp.bfloat16))
    )
    t3 = target[:ROW0].reshape(nb, rows_per_block, 1)

    out = pl.pallas_call(
        functools.partial(
            _tc_body, rows_per_block=rows_per_block, cpad=cpad
        ),
        grid=(nb,),
        in_specs=[
            pl.BlockSpec((1, rows_per_block, 1), lambda i: (i, 0, 0)),
            pl.BlockSpec((rows_per_block, d), lambda i: (i, 0)),
            pl.BlockSpec((cpad, d), lambda i: (0, 0)),
        ],
        out_specs=pl.BlockSpec(
            (1, 1), lambda i: (0, 0), memory_space=pltpu.SMEM
        ),
        out_shape=jax.ShapeDtypeStruct((1, 1), jnp.float32),
    )(t3, feature, w_bf)

    return out[0, 0]


def kernel(feature, target, weight):
    n, d = feature.shape
    sc_part = _sc_call(feature, target, weight)
    tc_part = _tc_call(feature, target, weight)
    return (tc_part + sc_part) / (n * d)


# speedup vs baseline: 1.9860x; 1.6653x over previous
import functools

import jax
import jax.numpy as jnp
from jax import lax
from jax.experimental import pallas as pl
from jax.experimental.pallas import tpu as pltpu


def _mse_body(t_ref, f_ref, w_ref, o_ref, *, rows_per_block, cpad):
    pid = pl.program_id(0)

    @pl.when(pid == 0)
    def _():
        o_ref[0, 0] = 0.0

    t_col = t_ref[0]
    iota = lax.broadcasted_iota(jnp.int32, (rows_per_block, cpad), 1)
    onehot = (t_col == iota).astype(jnp.bfloat16)
    g = jnp.dot(onehot, w_ref[...], preferred_element_type=jnp.float32)
    diff = f_ref[...] - g
    o_ref[0, 0] += jnp.sum(diff * diff)


def kernel(feature, target, weight):
    n, d = feature.shape
    c = weight.shape[0]
    cpad = 1024
    rows_per_block = 256
    nb = n // rows_per_block

    w_pad = (
        jnp.zeros((cpad, d), jnp.bfloat16)
        .at[:c]
        .set(weight.astype(jnp.bfloat16))
    )
    t3 = target.reshape(nb, rows_per_block, 1)

    out = pl.pallas_call(
        functools.partial(
            _mse_body, rows_per_block=rows_per_block, cpad=cpad
        ),
        grid=(nb,),
        in_specs=[
            pl.BlockSpec((1, rows_per_block, 1), lambda i: (i, 0, 0)),
            pl.BlockSpec((rows_per_block, d), lambda i: (i, 0)),
            pl.BlockSpec((cpad, d), lambda i: (0, 0)),
        ],
        out_specs=pl.BlockSpec(
            (1, 1), lambda i: (0, 0), memory_space=pltpu.SMEM
        ),
        out_shape=jax.ShapeDtypeStruct((1, 1), jnp.float32),
    )(t3, feature, w_pad)

    return out[0, 0] / (n * d)


# speedup vs baseline: 2.0821x; 1.0484x over previous
import functools

import jax
import jax.numpy as jnp
from jax import lax
from jax.experimental import pallas as pl
from jax.experimental.pallas import tpu as pltpu


def _mse_body(t_ref, f_ref, w_ref, o_ref, *, rows_per_block, cpad):
    pid = pl.program_id(0)

    @pl.when(pid == 0)
    def _():
        o_ref[0, 0] = 0.0

    t_col = t_ref[0]
    iota = lax.broadcasted_iota(jnp.int32, (rows_per_block, cpad), 1)
    onehot = (t_col == iota).astype(jnp.bfloat16)
    g = jnp.dot(onehot, w_ref[...], preferred_element_type=jnp.float32)
    diff = f_ref[...] - g
    o_ref[0, 0] += jnp.sum(diff * diff)


def kernel(feature, target, weight):
    n, d = feature.shape
    c = weight.shape[0]
    cpad = 1024
    rows_per_block = 1024
    nb = n // rows_per_block

    w_pad = (
        jnp.zeros((cpad, d), jnp.bfloat16)
        .at[:c]
        .set(weight.astype(jnp.bfloat16))
    )
    t3 = target.reshape(nb, rows_per_block, 1)

    out = pl.pallas_call(
        functools.partial(
            _mse_body, rows_per_block=rows_per_block, cpad=cpad
        ),
        grid=(nb,),
        in_specs=[
            pl.BlockSpec((1, rows_per_block, 1), lambda i: (i, 0, 0)),
            pl.BlockSpec((rows_per_block, d), lambda i: (i, 0)),
            pl.BlockSpec((cpad, d), lambda i: (0, 0)),
        ],
        out_specs=pl.BlockSpec(
            (1, 1), lambda i: (0, 0), memory_space=pltpu.SMEM
        ),
        out_shape=jax.ShapeDtypeStruct((1, 1), jnp.float32),
    )(t3, feature, w_pad)

    return out[0, 0] / (n * d)
